# baseline (speedup 1.0000x reference)
import functools

import jax
import jax.numpy as jnp
from jax.experimental import pallas as pl
from jax.experimental.pallas import tpu as pltpu


def _conv_stats_kernel(x_ref, w_ref, y_ref, stats_ref, *, H, W):
    x = x_ref[0].astype(jnp.bfloat16)
    hw, cin = x.shape
    xe = jnp.pad(x, ((W + 1, W + 1), (0, 0)))
    g = jax.lax.broadcasted_iota(jnp.int32, (hw + 2 * W + 2, 1), 0)
    gm = jnp.mod(g, W)
    x_0 = jnp.where(gm == 0, jnp.bfloat16(0), xe)
    x_2 = jnp.where(gm == 1, jnp.bfloat16(0), xe)
    span = hw + 2 * W
    patches = jnp.concatenate(
        [x_0[0:span], xe[1:span + 1], x_2[2:span + 2]], axis=1
    )
    y = jnp.dot(patches[0:hw], w_ref[0],
                preferred_element_type=jnp.float32)
    y += jnp.dot(patches[W:W + hw], w_ref[1],
                 preferred_element_type=jnp.float32)
    y += jnp.dot(patches[2 * W:2 * W + hw], w_ref[2],
                 preferred_element_type=jnp.float32)
    yb = y.astype(jnp.bfloat16)
    y_ref[0] = yb
    s = jnp.sum(y, axis=0, keepdims=True)
    ss = jnp.sum(jnp.square(yb), axis=0, keepdims=True,
                 dtype=jnp.float32)
    stats_ref[0] = jnp.concatenate([s, ss], axis=0)


def _bn_relu_kernel(y_ref, sc_ref, sh_ref, o_ref):
    y = y_ref[0].astype(jnp.float32)
    o_ref[0] = jnp.maximum(y * sc_ref[...] + sh_ref[...], 0.0).astype(o_ref.dtype)


@jax.jit
def kernel(x_nchw, w_oihw, gamma, beta):
    eps = 1e-5
    N, Cin, H, W = x_nchw.shape
    Cout, _, KH, KW = w_oihw.shape
    HW = H * W

    x_hwc = jnp.transpose(x_nchw, (0, 2, 3, 1)).reshape(N, HW, Cin)
    w_k = (
        jnp.transpose(w_oihw, (2, 3, 1, 0))
        .reshape(KH, KW * Cin, Cout)
        .astype(jnp.bfloat16)
    )

    conv_body = functools.partial(_conv_stats_kernel, H=H, W=W)
    flops1 = 2 * N * HW * KH * KW * Cin * Cout
    bytes1 = x_hwc.size * 4 + w_k.size * 2 + N * HW * Cout * 2 + N * 2 * Cout * 4
    y, stats = pl.pallas_call(
        conv_body,
        out_shape=(
            jax.ShapeDtypeStruct((N, HW, Cout), jnp.bfloat16),
            jax.ShapeDtypeStruct((N, 2, Cout), jnp.float32),
        ),
        grid=(N,),
        in_specs=[
            pl.BlockSpec((1, HW, Cin), lambda n: (n, 0, 0)),
            pl.BlockSpec((KH, KW * Cin, Cout), lambda n: (0, 0, 0)),
        ],
        out_specs=(
            pl.BlockSpec((1, HW, Cout), lambda n: (n, 0, 0)),
            pl.BlockSpec((1, 2, Cout), lambda n: (n, 0, 0)),
        ),
        compiler_params=pltpu.CompilerParams(
            dimension_semantics=("parallel",),
            vmem_limit_bytes=48 * 1024 * 1024,
        ),
        cost_estimate=pl.CostEstimate(
            flops=flops1, transcendentals=0, bytes_accessed=bytes1
        ),
    )(x_hwc, w_k)

    totals = jnp.sum(stats, axis=0)
    count = N * HW
    mean = totals[0] / count
    var = jnp.maximum(totals[1] / count - mean * mean, 0.0)
    scale = gamma.astype(jnp.float32) * jax.lax.rsqrt(var + eps)
    shift = beta.astype(jnp.float32) - mean * scale

    bytes2 = N * HW * Cout * (2 + 4) + 2 * Cout * 4
    out = pl.pallas_call(
        _bn_relu_kernel,
        out_shape=jax.ShapeDtypeStruct((N, HW, Cout), x_nchw.dtype),
        grid=(N,),
        in_specs=[
            pl.BlockSpec((1, HW, Cout), lambda n: (n, 0, 0)),
            pl.BlockSpec((1, Cout), lambda n: (0, 0)),
            pl.BlockSpec((1, Cout), lambda n: (0, 0)),
        ],
        out_specs=pl.BlockSpec((1, HW, Cout), lambda n: (n, 0, 0)),
        compiler_params=pltpu.CompilerParams(
            dimension_semantics=("parallel",),
            vmem_limit_bytes=32 * 1024 * 1024,
        ),
        cost_estimate=pl.CostEstimate(
            flops=2 * N * HW * Cout, transcendentals=0, bytes_accessed=bytes2
        ),
    )(y, scale.reshape(1, Cout), shift.reshape(1, Cout))

    return jnp.transpose(out.reshape(N, H, W, Cout), (0, 3, 1, 2))


# speedup vs baseline: 1.3238x; 1.3238x over previous
import functools

import jax
import jax.numpy as jnp
from jax.experimental import pallas as pl
from jax.experimental.pallas import tpu as pltpu


def _conv_stats_kernel(x_ref, w_ref, m_ref, y_ref, stats_ref, *, H, W):
    x = x_ref[0].astype(jnp.bfloat16)
    hw, cin = x.shape
    lo = W + 1
    xe = jnp.pad(x, ((lo, lo), (0, 0)))
    span = hw + 2 * W
    stack = jnp.concatenate(
        [xe[0:span], xe[1:span + 1], xe[2:span + 2]], axis=1
    )
    y = jnp.dot(stack[0:hw] * m_ref[...], w_ref[0],
                preferred_element_type=jnp.float32)
    y += jnp.dot(stack[W:W + hw] * m_ref[...], w_ref[1],
                 preferred_element_type=jnp.float32)
    y += jnp.dot(stack[2 * W:2 * W + hw] * m_ref[...], w_ref[2],
                 preferred_element_type=jnp.float32)
    yb = y.astype(jnp.bfloat16)
    y_ref[0] = yb
    s = jnp.sum(y, axis=0, keepdims=True)
    ss = jnp.sum(jnp.square(yb), axis=0, keepdims=True,
                 dtype=jnp.float32)
    stats_ref[0] = jnp.concatenate([s, ss], axis=0)


def _bn_relu_kernel(y_ref, sc_ref, sh_ref, o_ref):
    y = y_ref[...].astype(jnp.float32)
    o_ref[...] = jnp.maximum(y * sc_ref[...] + sh_ref[...], 0.0).astype(o_ref.dtype)


@jax.jit
def kernel(x_nchw, w_oihw, gamma, beta):
    eps = 1e-5
    N, Cin, H, W = x_nchw.shape
    Cout, _, KH, KW = w_oihw.shape
    HW = H * W

    x_hwc = jnp.transpose(x_nchw, (0, 2, 3, 1)).reshape(N, HW, Cin)
    w_k = (
        jnp.transpose(w_oihw, (2, 3, 1, 0))
        .reshape(KH, KW * Cin, Cout)
        .astype(jnp.bfloat16)
    )
    p_col = jnp.arange(HW, dtype=jnp.int32)[:, None] % W
    kw_grp = jnp.arange(KW * Cin, dtype=jnp.int32)[None, :] // Cin
    mask = jnp.where(
        ((kw_grp == 0) & (p_col == 0)) | ((kw_grp == KW - 1) & (p_col == W - 1)),
        jnp.bfloat16(0), jnp.bfloat16(1),
    )

    conv_body = functools.partial(_conv_stats_kernel, H=H, W=W)
    flops1 = 2 * N * HW * KH * KW * Cin * Cout
    bytes1 = x_hwc.size * 4 + w_k.size * 2 + N * HW * Cout * 2 + N * 2 * Cout * 4
    y, stats = pl.pallas_call(
        conv_body,
        out_shape=(
            jax.ShapeDtypeStruct((N, HW, Cout), jnp.bfloat16),
            jax.ShapeDtypeStruct((N, 2, Cout), jnp.float32),
        ),
        grid=(N,),
        in_specs=[
            pl.BlockSpec((1, HW, Cin), lambda n: (n, 0, 0)),
            pl.BlockSpec((KH, KW * Cin, Cout), lambda n: (0, 0, 0)),
            pl.BlockSpec((HW, KW * Cin), lambda n: (0, 0)),
        ],
        out_specs=(
            pl.BlockSpec((1, HW, Cout), lambda n: (n, 0, 0)),
            pl.BlockSpec((1, 2, Cout), lambda n: (n, 0, 0)),
        ),
        compiler_params=pltpu.CompilerParams(
            dimension_semantics=("parallel",),
            vmem_limit_bytes=48 * 1024 * 1024,
        ),
        cost_estimate=pl.CostEstimate(
            flops=flops1, transcendentals=0, bytes_accessed=bytes1
        ),
    )(x_hwc, w_k, mask)

    totals = jnp.sum(stats, axis=0)
    count = N * HW
    mean = totals[0] / count
    var = jnp.maximum(totals[1] / count - mean * mean, 0.0)
    scale = gamma.astype(jnp.float32) * jax.lax.rsqrt(var + eps)
    shift = beta.astype(jnp.float32) - mean * scale

    nb = 2 if N % 2 == 0 else 1
    bytes2 = N * HW * Cout * (2 + 4) + 2 * Cout * 4
    out = pl.pallas_call(
        _bn_relu_kernel,
        out_shape=jax.ShapeDtypeStruct((N, HW, Cout), x_nchw.dtype),
        grid=(N // nb,),
        in_specs=[
            pl.BlockSpec((nb, HW, Cout), lambda n: (n, 0, 0)),
            pl.BlockSpec((1, Cout), lambda n: (0, 0)),
            pl.BlockSpec((1, Cout), lambda n: (0, 0)),
        ],
        out_specs=pl.BlockSpec((nb, HW, Cout), lambda n: (n, 0, 0)),
        compiler_params=pltpu.CompilerParams(
            dimension_semantics=("parallel",),
            vmem_limit_bytes=32 * 1024 * 1024,
        ),
        cost_estimate=pl.CostEstimate(
            flops=2 * N * HW * Cout, transcendentals=0, bytes_accessed=bytes2
        ),
    )(y, scale.reshape(1, Cout), shift.reshape(1, Cout))

    return jnp.transpose(out.reshape(N, H, W, Cout), (0, 3, 1, 2))


# speedup vs baseline: 1.3692x; 1.0343x over previous
import functools

import jax
import jax.numpy as jnp
from jax.experimental import pallas as pl
from jax.experimental.pallas import tpu as pltpu


def _conv_stats_kernel(x_ref, w_ref, m_ref, y_ref, stats_ref, *, H, W):
    x = x_ref[0].astype(jnp.bfloat16)
    hw, cin = x.shape
    lo = W + 1
    xe = jnp.pad(x, ((lo, lo), (0, 0)))
    span = hw + 2 * W
    stack = jnp.concatenate(
        [xe[0:span], xe[1:span + 1], xe[2:span + 2]], axis=1
    ) * m_ref[...]
    y = jnp.dot(stack[0:hw], w_ref[0],
                preferred_element_type=jnp.float32)
    y += jnp.dot(stack[W:W + hw], w_ref[1],
                 preferred_element_type=jnp.float32)
    y += jnp.dot(stack[2 * W:2 * W + hw], w_ref[2],
                 preferred_element_type=jnp.float32)
    yb = y.astype(jnp.bfloat16)
    y_ref[0] = yb
    s = jnp.sum(y, axis=0, keepdims=True)
    ss = jnp.sum(jnp.square(yb), axis=0, keepdims=True,
                 dtype=jnp.float32)
    stats_ref[0] = jnp.concatenate([s, ss], axis=0)


def _bn_relu_kernel(y_ref, sc_ref, sh_ref, o_ref):
    y = y_ref[...].astype(jnp.float32)
    o_ref[...] = jnp.maximum(y * sc_ref[...] + sh_ref[...], 0.0).astype(o_ref.dtype)


@jax.jit
def kernel(x_nchw, w_oihw, gamma, beta):
    eps = 1e-5
    N, Cin, H, W = x_nchw.shape
    Cout, _, KH, KW = w_oihw.shape
    HW = H * W

    x_hwc = jnp.transpose(x_nchw, (0, 2, 3, 1)).reshape(N, HW, Cin)
    w_k = (
        jnp.transpose(w_oihw, (2, 3, 1, 0))
        .reshape(KH, KW * Cin, Cout)
        .astype(jnp.bfloat16)
    )
    span = HW + 2 * W
    p_col = jnp.arange(span, dtype=jnp.int32)[:, None] % W
    kw_grp = jnp.arange(KW * Cin, dtype=jnp.int32)[None, :] // Cin
    mask = jnp.where(
        ((kw_grp == 0) & (p_col == 0)) | ((kw_grp == KW - 1) & (p_col == W - 1)),
        jnp.bfloat16(0), jnp.bfloat16(1),
    )

    conv_body = functools.partial(_conv_stats_kernel, H=H, W=W)
    flops1 = 2 * N * HW * KH * KW * Cin * Cout
    bytes1 = x_hwc.size * 4 + w_k.size * 2 + N * HW * Cout * 2 + N * 2 * Cout * 4
    y, stats = pl.pallas_call(
        conv_body,
        out_shape=(
            jax.ShapeDtypeStruct((N, HW, Cout), jnp.bfloat16),
            jax.ShapeDtypeStruct((N, 2, Cout), jnp.float32),
        ),
        grid=(N,),
        in_specs=[
            pl.BlockSpec((1, HW, Cin), lambda n: (n, 0, 0)),
            pl.BlockSpec((KH, KW * Cin, Cout), lambda n: (0, 0, 0)),
            pl.BlockSpec((HW + 2 * W, KW * Cin), lambda n: (0, 0)),
        ],
        out_specs=(
            pl.BlockSpec((1, HW, Cout), lambda n: (n, 0, 0)),
            pl.BlockSpec((1, 2, Cout), lambda n: (n, 0, 0)),
        ),
        compiler_params=pltpu.CompilerParams(
            dimension_semantics=("parallel",),
            vmem_limit_bytes=48 * 1024 * 1024,
        ),
        cost_estimate=pl.CostEstimate(
            flops=flops1, transcendentals=0, bytes_accessed=bytes1
        ),
    )(x_hwc, w_k, mask)

    totals = jnp.sum(stats, axis=0)
    count = N * HW
    mean = totals[0] / count
    var = jnp.maximum(totals[1] / count - mean * mean, 0.0)
    scale = gamma.astype(jnp.float32) * jax.lax.rsqrt(var + eps)
    shift = beta.astype(jnp.float32) - mean * scale

    nb = 4 if N % 4 == 0 else 1
    bytes2 = N * HW * Cout * (2 + 4) + 2 * Cout * 4
    out = pl.pallas_call(
        _bn_relu_kernel,
        out_shape=jax.ShapeDtypeStruct((N, HW, Cout), x_nchw.dtype),
        grid=(N // nb,),
        in_specs=[
            pl.BlockSpec((nb, HW, Cout), lambda n: (n, 0, 0)),
            pl.BlockSpec((1, Cout), lambda n: (0, 0)),
            pl.BlockSpec((1, Cout), lambda n: (0, 0)),
        ],
        out_specs=pl.BlockSpec((nb, HW, Cout), lambda n: (n, 0, 0)),
        compiler_params=pltpu.CompilerParams(
            dimension_semantics=("parallel",),
            vmem_limit_bytes=32 * 1024 * 1024,
        ),
        cost_estimate=pl.CostEstimate(
            flops=2 * N * HW * Cout, transcendentals=0, bytes_accessed=bytes2
        ),
    )(y, scale.reshape(1, Cout), shift.reshape(1, Cout))

    return jnp.transpose(out.reshape(N, H, W, Cout), (0, 3, 1, 2))


# speedup vs baseline: 1.4250x; 1.0408x over previous
import functools

import jax
import jax.numpy as jnp
from jax.experimental import pallas as pl
from jax.experimental.pallas import tpu as pltpu


def _conv_stats_kernel(x_ref, w_ref, m_ref, y_ref, stats_ref, *, H, W):
    for i in range(x_ref.shape[0]):
        x = x_ref[i].astype(jnp.bfloat16)
        hw, cin = x.shape
        lo = W + 1
        xe = jnp.pad(x, ((lo, lo), (0, 0)))
        span = hw + 2 * W
        stack = jnp.concatenate(
            [xe[0:span], xe[1:span + 1], xe[2:span + 2]], axis=1
        ) * m_ref[...]
        y = jnp.dot(stack[0:hw], w_ref[0],
                    preferred_element_type=jnp.float32)
        y += jnp.dot(stack[W:W + hw], w_ref[1],
                     preferred_element_type=jnp.float32)
        y += jnp.dot(stack[2 * W:2 * W + hw], w_ref[2],
                     preferred_element_type=jnp.float32)
        yb = y.astype(jnp.bfloat16)
        y_ref[i] = yb
        s = jnp.sum(y, axis=0, keepdims=True)
        ss = jnp.sum(jnp.square(yb), axis=0, keepdims=True,
                     dtype=jnp.float32)
        stats_ref[i] = jnp.concatenate([s, ss], axis=0)


def _bn_relu_kernel(y_ref, sc_ref, sh_ref, o_ref):
    y = y_ref[...].astype(jnp.float32)
    o_ref[...] = jnp.maximum(y * sc_ref[...] + sh_ref[...], 0.0).astype(o_ref.dtype)


@jax.jit
def kernel(x_nchw, w_oihw, gamma, beta):
    eps = 1e-5
    N, Cin, H, W = x_nchw.shape
    Cout, _, KH, KW = w_oihw.shape
    HW = H * W

    x_hwc = jnp.transpose(x_nchw, (0, 2, 3, 1)).reshape(N, HW, Cin)
    w_k = (
        jnp.transpose(w_oihw, (2, 3, 1, 0))
        .reshape(KH, KW * Cin, Cout)
        .astype(jnp.bfloat16)
    )
    span = HW + 2 * W
    p_col = jnp.arange(span, dtype=jnp.int32)[:, None] % W
    kw_grp = jnp.arange(KW * Cin, dtype=jnp.int32)[None, :] // Cin
    mask = jnp.where(
        ((kw_grp == 0) & (p_col == 0)) | ((kw_grp == KW - 1) & (p_col == W - 1)),
        jnp.bfloat16(0), jnp.bfloat16(1),
    )

    nb1 = 2 if N % 2 == 0 else 1
    conv_body = functools.partial(_conv_stats_kernel, H=H, W=W)
    flops1 = 2 * N * HW * KH * KW * Cin * Cout
    bytes1 = x_hwc.size * 4 + w_k.size * 2 + N * HW * Cout * 2 + N * 2 * Cout * 4
    y, stats = pl.pallas_call(
        conv_body,
        out_shape=(
            jax.ShapeDtypeStruct((N, HW, Cout), jnp.bfloat16),
            jax.ShapeDtypeStruct((N, 2, Cout), jnp.float32),
        ),
        grid=(N // nb1,),
        in_specs=[
            pl.BlockSpec((nb1, HW, Cin), lambda n: (n, 0, 0)),
            pl.BlockSpec((KH, KW * Cin, Cout), lambda n: (0, 0, 0)),
            pl.BlockSpec((HW + 2 * W, KW * Cin), lambda n: (0, 0)),
        ],
        out_specs=(
            pl.BlockSpec((nb1, HW, Cout), lambda n: (n, 0, 0)),
            pl.BlockSpec((nb1, 2, Cout), lambda n: (n, 0, 0)),
        ),
        compiler_params=pltpu.CompilerParams(
            dimension_semantics=("parallel",),
            vmem_limit_bytes=48 * 1024 * 1024,
        ),
        cost_estimate=pl.CostEstimate(
            flops=flops1, transcendentals=0, bytes_accessed=bytes1
        ),
    )(x_hwc, w_k, mask)

    totals = jnp.sum(stats, axis=0)
    count = N * HW
    mean = totals[0] / count
    var = jnp.maximum(totals[1] / count - mean * mean, 0.0)
    scale = gamma.astype(jnp.float32) * jax.lax.rsqrt(var + eps)
    shift = beta.astype(jnp.float32) - mean * scale

    nb = 4 if N % 4 == 0 else 1
    bytes2 = N * HW * Cout * (2 + 4) + 2 * Cout * 4
    out = pl.pallas_call(
        _bn_relu_kernel,
        out_shape=jax.ShapeDtypeStruct((N, HW, Cout), x_nchw.dtype),
        grid=(N // nb,),
        in_specs=[
            pl.BlockSpec((nb, HW, Cout), lambda n: (n, 0, 0)),
            pl.BlockSpec((1, Cout), lambda n: (0, 0)),
            pl.BlockSpec((1, Cout), lambda n: (0, 0)),
        ],
        out_specs=pl.BlockSpec((nb, HW, Cout), lambda n: (n, 0, 0)),
        compiler_params=pltpu.CompilerParams(
            dimension_semantics=("parallel",),
            vmem_limit_bytes=32 * 1024 * 1024,
        ),
        cost_estimate=pl.CostEstimate(
            flops=2 * N * HW * Cout, transcendentals=0, bytes_accessed=bytes2
        ),
    )(y, scale.reshape(1, Cout), shift.reshape(1, Cout))

    return jnp.transpose(out.reshape(N, H, W, Cout), (0, 3, 1, 2))


# speedup vs baseline: 1.4327x; 1.0054x over previous
import functools

import jax
import jax.numpy as jnp
import numpy as np
from jax.experimental import pallas as pl
from jax.experimental.pallas import tpu as pltpu


def _conv_stats_kernel(x_ref, w_ref, m_ref, y_ref, stats_ref, *, H, W):
    for i in range(x_ref.shape[0]):
        x = x_ref[i].astype(jnp.bfloat16)
        hw, cin = x.shape
        lo = W + 1
        xe = jnp.pad(x, ((lo, lo), (0, 0)))
        span = hw + 2 * W
        stack = jnp.concatenate(
            [xe[0:span], xe[1:span + 1], xe[2:span + 2]], axis=1
        ) * m_ref[...]
        y = jnp.dot(stack[0:hw], w_ref[0],
                    preferred_element_type=jnp.float32)
        y += jnp.dot(stack[W:W + hw], w_ref[1],
                     preferred_element_type=jnp.float32)
        y += jnp.dot(stack[2 * W:2 * W + hw], w_ref[2],
                     preferred_element_type=jnp.float32)
        yb = y.astype(jnp.bfloat16)
        y_ref[i] = yb
        s = jnp.sum(y, axis=0, keepdims=True)
        ss = jnp.sum(jnp.square(yb), axis=0, keepdims=True,
                     dtype=jnp.float32)
        stats_ref[i] = jnp.concatenate([s, ss], axis=0)


def _bn_relu_kernel(y_ref, st_ref, g_ref, b_ref, o_ref, *, count, eps):
    totals = jnp.sum(st_ref[...], axis=0)
    mean = totals[0:1] / count
    var = jnp.maximum(totals[1:2] / count - mean * mean, 0.0)
    scale = g_ref[...] * jax.lax.rsqrt(var + eps)
    shift = b_ref[...] - mean * scale
    y = y_ref[...].astype(jnp.float32)
    o_ref[...] = jnp.maximum(y * scale + shift, 0.0).astype(o_ref.dtype)


@jax.jit
def kernel(x_nchw, w_oihw, gamma, beta):
    eps = 1e-5
    N, Cin, H, W = x_nchw.shape
    Cout, _, KH, KW = w_oihw.shape
    HW = H * W

    x_hwc = jnp.transpose(x_nchw, (0, 2, 3, 1)).reshape(N, HW, Cin)
    w_k = (
        jnp.transpose(w_oihw, (2, 3, 1, 0))
        .reshape(KH, KW * Cin, Cout)
        .astype(jnp.bfloat16)
    )
    span = HW + 2 * W
    p_col = np.arange(span)[:, None] % W
    kw_grp = np.arange(KW * Cin)[None, :] // Cin
    mask = jnp.asarray(np.where(
        ((kw_grp == 0) & (p_col == 0)) | ((kw_grp == KW - 1) & (p_col == W - 1)),
        0.0, 1.0,
    ), dtype=jnp.bfloat16)

    nb1 = 2 if N % 2 == 0 else 1
    conv_body = functools.partial(_conv_stats_kernel, H=H, W=W)
    flops1 = 2 * N * HW * KH * KW * Cin * Cout
    bytes1 = x_hwc.size * 4 + w_k.size * 2 + N * HW * Cout * 2 + N * 2 * Cout * 4
    y, stats = pl.pallas_call(
        conv_body,
        out_shape=(
            jax.ShapeDtypeStruct((N, HW, Cout), jnp.bfloat16),
            jax.ShapeDtypeStruct((N, 2, Cout), jnp.float32),
        ),
        grid=(N // nb1,),
        in_specs=[
            pl.BlockSpec((nb1, HW, Cin), lambda n: (n, 0, 0)),
            pl.BlockSpec((KH, KW * Cin, Cout), lambda n: (0, 0, 0)),
            pl.BlockSpec((HW + 2 * W, KW * Cin), lambda n: (0, 0)),
        ],
        out_specs=(
            pl.BlockSpec((nb1, HW, Cout), lambda n: (n, 0, 0)),
            pl.BlockSpec((nb1, 2, Cout), lambda n: (n, 0, 0)),
        ),
        compiler_params=pltpu.CompilerParams(
            dimension_semantics=("parallel",),
            vmem_limit_bytes=48 * 1024 * 1024,
        ),
        cost_estimate=pl.CostEstimate(
            flops=flops1, transcendentals=0, bytes_accessed=bytes1
        ),
    )(x_hwc, w_k, mask)

    nb = 4 if N % 4 == 0 else 1
    bn_body = functools.partial(_bn_relu_kernel, count=float(N * HW), eps=eps)
    bytes2 = N * HW * Cout * (2 + 4) + (N + 1) * 2 * Cout * 4
    out = pl.pallas_call(
        bn_body,
        out_shape=jax.ShapeDtypeStruct((N, HW, Cout), x_nchw.dtype),
        grid=(N // nb,),
        in_specs=[
            pl.BlockSpec((nb, HW, Cout), lambda n: (n, 0, 0)),
            pl.BlockSpec((N, 2, Cout), lambda n: (0, 0, 0)),
            pl.BlockSpec((1, Cout), lambda n: (0, 0)),
            pl.BlockSpec((1, Cout), lambda n: (0, 0)),
        ],
        out_specs=pl.BlockSpec((nb, HW, Cout), lambda n: (n, 0, 0)),
        compiler_params=pltpu.CompilerParams(
            dimension_semantics=("parallel",),
            vmem_limit_bytes=32 * 1024 * 1024,
        ),
        cost_estimate=pl.CostEstimate(
            flops=2 * N * HW * Cout, transcendentals=0, bytes_accessed=bytes2
        ),
    )(y, stats,
      gamma.astype(jnp.float32).reshape(1, Cout),
      beta.astype(jnp.float32).reshape(1, Cout))

    return jnp.transpose(out.reshape(N, H, W, Cout), (0, 3, 1, 2))


# speedup vs baseline: 1.4408x; 1.0056x over previous
import functools

import jax
import jax.numpy as jnp
import numpy as np
from jax.experimental import pallas as pl
from jax.experimental.pallas import tpu as pltpu


def _conv_stats_kernel(x_ref, w_ref, m_ref, y_ref, stats_ref, *, H, W):
    for i in range(x_ref.shape[0]):
        x = x_ref[i].astype(jnp.bfloat16)
        hw, cin = x.shape
        lo = W + 1
        xe = jnp.pad(x, ((lo, lo), (0, 0)))
        span = hw + 2 * W
        stack = jnp.concatenate(
            [xe[0:span], xe[1:span + 1], xe[2:span + 2]], axis=1
        ) * m_ref[...]
        y = jnp.dot(stack[0:hw], w_ref[0],
                    preferred_element_type=jnp.float32)
        y += jnp.dot(stack[W:W + hw], w_ref[1],
                     preferred_element_type=jnp.float32)
        y += jnp.dot(stack[2 * W:2 * W + hw], w_ref[2],
                     preferred_element_type=jnp.float32)
        yb = y.astype(jnp.bfloat16)
        y_ref[i] = yb
        s = jnp.sum(y, axis=0, keepdims=True)
        ss = jnp.sum(jnp.square(yb), axis=0, keepdims=True,
                     dtype=jnp.float32)
        stats_ref[i] = jnp.concatenate([s, ss], axis=0)


def _bn_relu_kernel(y_ref, st_ref, g_ref, b_ref, o_ref, *, count, eps):
    totals = jnp.sum(st_ref[...], axis=0)
    mean = totals[0:1] / count
    var = jnp.maximum(totals[1:2] / count - mean * mean, 0.0)
    scale = g_ref[...] * jax.lax.rsqrt(var + eps)
    shift = b_ref[...] - mean * scale
    y = y_ref[...].astype(jnp.float32)
    o_ref[...] = jnp.maximum(y * scale + shift, 0.0).astype(o_ref.dtype)


@jax.jit
def kernel(x_nchw, w_oihw, gamma, beta):
    eps = 1e-5
    N, Cin, H, W = x_nchw.shape
    Cout, _, KH, KW = w_oihw.shape
    HW = H * W

    x_hwc = jnp.transpose(x_nchw, (0, 2, 3, 1)).reshape(N, HW, Cin)
    w_k = (
        jnp.transpose(w_oihw, (2, 3, 1, 0))
        .reshape(KH, KW * Cin, Cout)
        .astype(jnp.bfloat16)
    )
    span = HW + 2 * W
    p_col = np.arange(span)[:, None] % W
    kw_grp = np.arange(KW * Cin)[None, :] // Cin
    mask = jnp.asarray(np.where(
        ((kw_grp == 0) & (p_col == 0)) | ((kw_grp == KW - 1) & (p_col == W - 1)),
        0.0, 1.0,
    ), dtype=jnp.bfloat16)

    nb1 = 4 if N % 4 == 0 else 1
    conv_body = functools.partial(_conv_stats_kernel, H=H, W=W)
    flops1 = 2 * N * HW * KH * KW * Cin * Cout
    bytes1 = x_hwc.size * 4 + w_k.size * 2 + N * HW * Cout * 2 + N * 2 * Cout * 4
    y, stats = pl.pallas_call(
        conv_body,
        out_shape=(
            jax.ShapeDtypeStruct((N, HW, Cout), jnp.bfloat16),
            jax.ShapeDtypeStruct((N, 2, Cout), jnp.float32),
        ),
        grid=(N // nb1,),
        in_specs=[
            pl.BlockSpec((nb1, HW, Cin), lambda n: (n, 0, 0)),
            pl.BlockSpec((KH, KW * Cin, Cout), lambda n: (0, 0, 0)),
            pl.BlockSpec((HW + 2 * W, KW * Cin), lambda n: (0, 0)),
        ],
        out_specs=(
            pl.BlockSpec((nb1, HW, Cout), lambda n: (n, 0, 0)),
            pl.BlockSpec((nb1, 2, Cout), lambda n: (n, 0, 0)),
        ),
        compiler_params=pltpu.CompilerParams(
            dimension_semantics=("parallel",),
            vmem_limit_bytes=48 * 1024 * 1024,
        ),
        cost_estimate=pl.CostEstimate(
            flops=flops1, transcendentals=0, bytes_accessed=bytes1
        ),
    )(x_hwc, w_k, mask)

    nb = 8 if N % 8 == 0 else 1
    bn_body = functools.partial(_bn_relu_kernel, count=float(N * HW), eps=eps)
    bytes2 = N * HW * Cout * (2 + 4) + (N + 1) * 2 * Cout * 4
    out = pl.pallas_call(
        bn_body,
        out_shape=jax.ShapeDtypeStruct((N, HW, Cout), x_nchw.dtype),
        grid=(N // nb,),
        in_specs=[
            pl.BlockSpec((nb, HW, Cout), lambda n: (n, 0, 0)),
            pl.BlockSpec((N, 2, Cout), lambda n: (0, 0, 0)),
            pl.BlockSpec((1, Cout), lambda n: (0, 0)),
            pl.BlockSpec((1, Cout), lambda n: (0, 0)),
        ],
        out_specs=pl.BlockSpec((nb, HW, Cout), lambda n: (n, 0, 0)),
        compiler_params=pltpu.CompilerParams(
            dimension_semantics=("parallel",),
            vmem_limit_bytes=56 * 1024 * 1024,
        ),
        cost_estimate=pl.CostEstimate(
            flops=2 * N * HW * Cout, transcendentals=0, bytes_accessed=bytes2
        ),
    )(y, stats,
      gamma.astype(jnp.float32).reshape(1, Cout),
      beta.astype(jnp.float32).reshape(1, Cout))

    return jnp.transpose(out.reshape(N, H, W, Cout), (0, 3, 1, 2))
